# Initial kernel scaffold; baseline (speedup 1.0000x reference)
import jax
import jax.numpy as jnp
from jax.experimental import pallas as pl


def kernel(data, perm):
    raise NotImplementedError("write your pallas kernel here")


# speedup vs baseline: 1.4831x; 1.4831x over previous
import functools

import jax
import jax.numpy as jnp
from jax import lax
from jax.experimental import pallas as pl
from jax.experimental.pallas import tpu as pltpu
from jax.experimental.pallas import tpu_sc as plsc

B = 4
S = 8192
D = 1024
R = B * S
NC = 2
NS = 16
NW = NC * NS
RPW = R // NW
C = 32
NCHUNK = RPW // C
L = 16

_mesh = plsc.VectorSubcoreMesh(core_axis_name="c", subcore_axis_name="s")


@functools.partial(
    pl.kernel,
    mesh=_mesh,
    out_type=jax.ShapeDtypeStruct((R, D), jnp.float32),
    scratch_types=[
        pltpu.VMEM((RPW,), jnp.int32),
        pltpu.VMEM((C, D), jnp.float32),
        pltpu.SemaphoreType.DMA,
    ],
)
def _gather_rows(data_hbm, perm_hbm, out_hbm, idx_v, rows_v, sem):
    wid = lax.axis_index("s") * NC + lax.axis_index("c")
    base = wid * RPW

    pltpu.sync_copy(perm_hbm.at[pl.ds(base, RPW)], idx_v)
    row_off = (wid // (S // RPW)) * S

    def _add_off(i, carry):
        sl = pl.ds(i * L, L)
        idx_v[sl] = idx_v[sl] + row_off
        return carry

    lax.fori_loop(0, RPW // L, _add_off, 0)

    def _chunk(c, carry):
        idx_chunk = idx_v.at[pl.ds(c * C, C)]
        pltpu.async_copy(data_hbm.at[idx_chunk], rows_v, sem).wait()
        pltpu.sync_copy(rows_v, out_hbm.at[pl.ds(base + c * C, C)])
        return carry

    lax.fori_loop(0, NCHUNK, _chunk, 0)


def kernel(data, perm):
    out = _gather_rows(data.reshape(R, D), perm.reshape(R))
    return out.reshape(B, S, D)


# speedup vs baseline: 1.7290x; 1.1658x over previous
import functools

import jax
import jax.numpy as jnp
from jax import lax
from jax.experimental import pallas as pl
from jax.experimental.pallas import tpu as pltpu
from jax.experimental.pallas import tpu_sc as plsc

B = 4
S = 8192
D = 1024
R = B * S
NC = 2
NS = 16
NW = NC * NS
RPW = R // NW
C = 32
NCHUNK = RPW // C
L = 16

_mesh = plsc.VectorSubcoreMesh(core_axis_name="c", subcore_axis_name="s")


@functools.partial(
    pl.kernel,
    mesh=_mesh,
    out_type=jax.ShapeDtypeStruct((R, D), jnp.float32),
    scratch_types=[
        pltpu.VMEM((RPW,), jnp.int32),
        pltpu.VMEM((2 * C, D), jnp.float32),
        pltpu.SemaphoreType.DMA,
        pltpu.SemaphoreType.DMA,
    ],
)
def _gather_rows(data_hbm, perm_hbm, out_hbm, idx_v, rows_v, gsem, ssem):
    wid = lax.axis_index("s") * NC + lax.axis_index("c")
    base = wid * RPW

    pltpu.sync_copy(perm_hbm.at[pl.ds(base, RPW)], idx_v)
    row_off = (wid // (S // RPW)) * S

    def _add_off(i, carry):
        sl = pl.ds(i * L, L)
        idx_v[sl] = idx_v[sl] + row_off
        return carry

    lax.fori_loop(0, RPW // L, _add_off, 0)

    def _buf(b):
        return rows_v.at[pl.ds(b * C, C)]

    def _gather(c, b):
        return pltpu.async_copy(data_hbm.at[idx_v.at[pl.ds(c * C, C)]],
                                _buf(b), gsem)

    def _scatter(c, b):
        return pltpu.async_copy(_buf(b), out_hbm.at[pl.ds(base + c * C, C)],
                                ssem)

    gd = [None] * NCHUNK
    sd = [None] * NCHUNK
    for c in range(NCHUNK):
        b = c % 2
        if c >= 2:
            sd[c - 2].wait()
        gd[c] = _gather(c, b)
        if c >= 1:
            gd[c - 1].wait()
            sd[c - 1] = _scatter(c - 1, 1 - b)
    last = NCHUNK - 1
    gd[last].wait()
    sd[last] = _scatter(last, last % 2)
    sd[last - 1].wait()
    sd[last].wait()


def kernel(data, perm):
    out = _gather_rows(data.reshape(R, D), perm.reshape(R))
    return out.reshape(B, S, D)


# speedup vs baseline: 1.8004x; 1.0413x over previous
import functools

import jax
import jax.numpy as jnp
from jax import lax
from jax.experimental import pallas as pl
from jax.experimental.pallas import tpu as pltpu
from jax.experimental.pallas import tpu_sc as plsc

B = 4
S = 8192
D = 1024
R = B * S
NC = 2
NS = 16
NW = NC * NS
RPW = R // NW
C = 32
NCHUNK = RPW // C
L = 16

_mesh = plsc.VectorSubcoreMesh(core_axis_name="c", subcore_axis_name="s")


NBUF = 3


@functools.partial(
    pl.kernel,
    mesh=_mesh,
    out_type=jax.ShapeDtypeStruct((R, D), jnp.float32),
    scratch_types=[
        pltpu.VMEM((RPW,), jnp.int32),
        pltpu.VMEM((NBUF * C, D), jnp.float32),
        pltpu.SemaphoreType.DMA,
        pltpu.SemaphoreType.DMA,
    ],
)
def _gather_rows(data_hbm, perm_hbm, out_hbm, idx_v, rows_v, gsem, ssem):
    wid = lax.axis_index("s") * NC + lax.axis_index("c")
    base = wid * RPW

    pltpu.sync_copy(perm_hbm.at[pl.ds(base, RPW)], idx_v)
    row_off = (wid // (S // RPW)) * S

    def _add_off(i, carry):
        sl = pl.ds(i * L, L)
        idx_v[sl] = idx_v[sl] + row_off
        return carry

    lax.fori_loop(0, RPW // L, _add_off, 0)

    def _buf(b):
        return rows_v.at[pl.ds(b * C, C)]

    def _gdesc(c, b):
        return pltpu.make_async_copy(data_hbm.at[idx_v.at[pl.ds(c * C, C)]],
                                     _buf(b), gsem)

    def _sdesc(c, b):
        return pltpu.make_async_copy(_buf(b),
                                     out_hbm.at[pl.ds(base + c * C, C)], ssem)

    def _step(c, carry):
        b = lax.rem(c, NBUF)

        @pl.when(c >= NBUF)
        def _wait_scatter():
            _sdesc(c - NBUF, b).wait()

        _gdesc(c, b).start()

        @pl.when(c >= 1)
        def _drain_prev():
            pb = lax.rem(c - 1, NBUF)
            _gdesc(c - 1, pb).wait()
            _sdesc(c - 1, pb).start()

        return carry

    lax.fori_loop(0, NCHUNK, _step, 0)

    last = NCHUNK - 1
    lb = last % NBUF
    _gdesc(last, lb).wait()
    _sdesc(last, lb).start()

    def _drain(i, carry):
        c = NCHUNK - NBUF + i
        _sdesc(c, lax.rem(c, NBUF)).wait()
        return carry

    lax.fori_loop(0, NBUF, _drain, 0)


def kernel(data, perm):
    out = _gather_rows(data.reshape(R, D), perm.reshape(R))
    return out.reshape(B, S, D)
